# baseline (speedup 1.0000x reference)
import math

import jax
import jax.numpy as jnp
from jax import lax
from jax.experimental import pallas as pl
from jax.experimental.pallas import tpu as pltpu
from jax.experimental.pallas import tpu_sc as plsc

N_NODES = 10000
N_EDGES = 160000
D = 128
NPAD = 10112
CH = 16
NC, NS, L = 2, 16, 16
NW = NC * NS
NEPAD = 161280
NCHUNK = NEPAD // CH
NT = NCHUNK // NW
BN_SCALE = float(1.0 / math.sqrt(1.0 + 1e-5))
EPS = 1e-5

NB = 1000
EB = 4032


def _leaky(x):
    return jnp.maximum(x, 0.1 * x)


def _dot(a, b):
    return lax.dot_general(a, b, (((1,), (0,)), ((), ())),
                           preferred_element_type=jnp.float32)


def _full(shape):
    return pl.BlockSpec(shape, lambda i: (0,) * len(shape))


def _enc2(x, w1, b1, g1, be1, w2, b2, g2, be2):
    h = _dot(x, w1) + b1
    h = _leaky(h * (g1 * BN_SCALE) + be1)
    h = _dot(h, w2) + b2
    return _leaky(h * (g2 * BN_SCALE) + be2)


def _node_enc_body(x_ref, w1, b1, g1, be1, w2, b2, g2, be2, w1s, w1d,
                   emb_ref, ab_ref):
    h = _enc2(x_ref[...], w1[...], b1[...], g1[...], be1[...],
              w2[...], b2[...], g2[...], be2[...])
    emb_ref[...] = h
    ab_ref[:, :D] = _dot(h, w1s[...])
    ab_ref[:, D:] = _dot(h, w1d[...])


def _node_enc_call(x, ne, w1s, w1d):
    grid = (N_NODES // NB,)
    return pl.pallas_call(
        _node_enc_body,
        grid=grid,
        in_specs=[pl.BlockSpec((NB, D), lambda i: (i, 0)),
                  _full((D, D)), _full((1, D)), _full((1, D)), _full((1, D)),
                  _full((D, D)), _full((1, D)), _full((1, D)), _full((1, D)),
                  _full((D, D)), _full((D, D))],
        out_specs=[pl.BlockSpec((NB, D), lambda i: (i, 0)),
                   pl.BlockSpec((NB, 2 * D), lambda i: (i, 0))],
        out_shape=[jax.ShapeDtypeStruct((N_NODES, D), jnp.float32),
                   jax.ShapeDtypeStruct((N_NODES, 2 * D), jnp.float32)],
    )(x, ne["W1"], ne["b1"].reshape(1, D), ne["g1"].reshape(1, D),
      ne["be1"].reshape(1, D), ne["W2"], ne["b2"].reshape(1, D),
      ne["g2"].reshape(1, D), ne["be2"].reshape(1, D), w1s, w1d)


def _edge_enc_body(x_ref, w1, b1, g1, be1, w2, b2, g2, be2,
                   w1e1, c1, w1e2, c2, e1_ref, e2_ref):
    h = _enc2(x_ref[...], w1[...], b1[...], g1[...], be1[...],
              w2[...], b2[...], g2[...], be2[...])
    e1_ref[...] = _dot(h, w1e1[...]) + c1[...]
    e2_ref[...] = _dot(h, w1e2[...]) + c2[...]


def _edge_enc_call(ef, ee, w1e1, c1, w1e2, c2):
    ed = ef.shape[1]
    ne = ef.shape[0]
    grid = (ne // EB,)
    return pl.pallas_call(
        _edge_enc_body,
        grid=grid,
        in_specs=[pl.BlockSpec((EB, ed), lambda i: (i, 0)),
                  _full((ed, D)), _full((1, D)), _full((1, D)), _full((1, D)),
                  _full((D, D)), _full((1, D)), _full((1, D)), _full((1, D)),
                  _full((D, D)), _full((1, D)), _full((D, D)), _full((1, D))],
        out_specs=[pl.BlockSpec((EB, D), lambda i: (i, 0))] * 2,
        out_shape=[jax.ShapeDtypeStruct((ne, D), jnp.float32)] * 2,
    )(ef, ee["W1"], ee["b1"].reshape(1, D), ee["g1"].reshape(1, D),
      ee["be1"].reshape(1, D), ee["W2"], ee["b2"].reshape(1, D),
      ee["g2"].reshape(1, D), ee["be2"].reshape(1, D),
      w1e1, c1.reshape(1, D), w1e2, c2.reshape(1, D))


def _combine_body_proj(emb_ref, s_ref, deg_ref, w2m, b2m, wa1, ba1, lg, lb,
                       wa2, ba2, w1s, w1d, out_ref, ab_ref):
    e = emb_ref[...]
    s = s_ref[0, :, :] + s_ref[1, :, :]
    deg = deg_ref[0, :, :] + deg_ref[1, :, :]
    new = e + _dot(s, w2m[...]) + deg * b2m[...]
    g = _dot(new, wa1[...]) + ba1[...]
    mu = jnp.mean(g, axis=-1, keepdims=True)
    var = jnp.mean((g - mu) ** 2, axis=-1, keepdims=True)
    h = _leaky((g - mu) * lax.rsqrt(var + EPS) * lg[...] + lb[...])
    out = e + _dot(h, wa2[...]) + ba2[...]
    out_ref[...] = out
    ab_ref[:, :D] = _dot(out, w1s[...])
    ab_ref[:, D:] = _dot(out, w1d[...])


def _combine_body_final(emb_ref, s_ref, deg_ref, w2m, b2m, wa1, ba1, lg, lb,
                        wa2, ba2, out_ref):
    e = emb_ref[...]
    s = s_ref[0, :, :] + s_ref[1, :, :]
    deg = deg_ref[0, :, :] + deg_ref[1, :, :]
    new = e + _dot(s, w2m[...]) + deg * b2m[...]
    g = _dot(new, wa1[...]) + ba1[...]
    mu = jnp.mean(g, axis=-1, keepdims=True)
    var = jnp.mean((g - mu) ** 2, axis=-1, keepdims=True)
    h = _leaky((g - mu) * lax.rsqrt(var + EPS) * lg[...] + lb[...])
    out_ref[...] = e + _dot(h, wa2[...]) + ba2[...]


def _combine_call(emb, S, degp, mp, ag, proj):
    grid = (N_NODES // NB,)
    in_specs = [pl.BlockSpec((NB, D), lambda i: (i, 0)),
                pl.BlockSpec((2, NB, D), lambda i: (0, i, 0)),
                pl.BlockSpec((2, NB, 1), lambda i: (0, i, 0)),
                _full((D, D)), _full((1, D)),
                _full((D, D)), _full((1, D)), _full((1, D)), _full((1, D)),
                _full((D, D)), _full((1, D))]
    args = [emb, S, degp, mp["W2"], mp["b2"].reshape(1, D),
            ag["W1"], ag["b1"].reshape(1, D), ag["lng"].reshape(1, D),
            ag["lnb"].reshape(1, D), ag["W2"], ag["b2"].reshape(1, D)]
    if proj is not None:
        w1s, w1d = proj
        return pl.pallas_call(
            _combine_body_proj,
            grid=grid,
            in_specs=in_specs + [_full((D, D)), _full((D, D))],
            out_specs=[pl.BlockSpec((NB, D), lambda i: (i, 0)),
                       pl.BlockSpec((NB, 2 * D), lambda i: (i, 0))],
            out_shape=[jax.ShapeDtypeStruct((N_NODES, D), jnp.float32),
                       jax.ShapeDtypeStruct((N_NODES, 2 * D), jnp.float32)],
        )(*args, w1s, w1d)
    return pl.pallas_call(
        _combine_body_final,
        grid=grid,
        in_specs=in_specs,
        out_specs=pl.BlockSpec((NB, D), lambda i: (i, 0)),
        out_shape=jax.ShapeDtypeStruct((N_NODES, D), jnp.float32),
    )(*args)


_GDN = lax.GatherDimensionNumbers(offset_dims=(), collapsed_slice_dims=(0,),
                                  start_index_map=(0,))


def _perm16(v, idx):
    return lax.gather(v, idx[:, None], _GDN, slice_sizes=(1,),
                      mode=lax.GatherScatterMode.PROMISE_IN_BOUNDS)


def _hsum16(v):
    lanes = lax.iota(jnp.int32, 16)
    for sh in (8, 4, 2, 1):
        v = v + _perm16(v, lanes ^ sh)
    return v


def _rsqrt_vec(x):
    i = lax.bitcast_convert_type(x, jnp.int32)
    y = lax.bitcast_convert_type(jnp.int32(0x5F3759DF) - (i >> 1), jnp.float32)
    for _ in range(2):
        y = y * (1.5 - 0.5 * x * y * y)
    return y


def _make_msg_kernel(with_deg):
    mesh = plsc.VectorSubcoreMesh(core_axis_name="c", subcore_axis_name="s")
    out_type = [jax.ShapeDtypeStruct((NC, NPAD, D), jnp.float32)]
    if with_deg:
        out_type.append(jax.ShapeDtypeStruct((NC, NPAD), jnp.float32))
    scratch = [
        pltpu.VMEM((2, CH), jnp.int32),
        pltpu.VMEM((2, CH), jnp.int32),
        pltpu.VMEM((2, CH), jnp.int32),
        pltpu.VMEM((CH, 2 * D), jnp.float32),
        pltpu.VMEM((CH, 2 * D), jnp.float32),
        pltpu.VMEM((CH, 2 * D), jnp.float32),
        pltpu.VMEM((CH, 2 * D), jnp.float32),
        pltpu.VMEM((CH, 2 * D), jnp.float32),
        pltpu.VMEM((CH, 2 * D), jnp.float32),
        pltpu.VMEM((2 * CH, D), jnp.float32),
        pltpu.VMEM((2 * CH, D), jnp.float32),
        pltpu.VMEM((2 * CH, D), jnp.float32),
        pltpu.VMEM((CH,), jnp.float32),
        pltpu.VMEM((D,), jnp.float32),
        pltpu.VMEM((D,), jnp.float32),
        pltpu.VMEM_SHARED((NPAD, D), jnp.float32),
        pltpu.VMEM_SHARED((NPAD,), jnp.float32),
        pltpu.SemaphoreType.DMA,
        pltpu.SemaphoreType.DMA,
        pltpu.SemaphoreType.DMA,
        pltpu.SemaphoreType.DMA,
        pltpu.SemaphoreType.DMA,
        pltpu.SemaphoreType.DMA,
        pltpu.SemaphoreType.DMA,
        pltpu.SemaphoreType.DMA,
        pltpu.SemaphoreType.DMA,
    ]

    def body(src, dst, g_hbm, e_hbm, lg_hbm, lb_hbm, s_out, *rest):
        if with_deg:
            deg_out = rest[0]
            rest = rest[1:]
        else:
            deg_out = None
        (sidi0, sidi1, sidi2, gsi0, gsi1, gsi2, gdi0, gdi1, gdi2,
         hbig0, hbig1, hbig2, ones, lg, lb, acc, degsh,
         im0, im1, im2, lm0, lm1, lm2, sm0, sm1, sm2) = rest
        sets = [(sidi0, gsi0, gdi0, hbig0, im0, lm0, sm0),
                (sidi1, gsi1, gdi1, hbig1, im1, lm1, sm1),
                (sidi2, gsi2, gdi2, hbig2, im2, lm2, sm2)]
        c = lax.axis_index("c")
        s = lax.axis_index("s")
        w = s * NC + c
        start = w * NT

        pltpu.sync_copy(lg_hbm, lg)
        pltpu.sync_copy(lb_hbm, lb)

        def zrow(r, carry):
            for k in range(D // L):
                hbig0[r, pl.ds(k * L, L)] = jnp.zeros((L,), jnp.float32)
            return carry
        lax.fori_loop(0, 2 * CH, zrow, 0)
        for k in range(CH // L):
            ones[pl.ds(k * L, L)] = jnp.ones((L,), jnp.float32)

        rows_per_tile = NPAD // NS
        base = s * rows_per_tile
        nfull = rows_per_tile // (2 * CH)
        rem = rows_per_tile - nfull * 2 * CH
        for j in range(nfull):
            pltpu.sync_copy(hbig0, acc.at[pl.ds(base + j * 2 * CH, 2 * CH)])
        pltpu.sync_copy(hbig0.at[pl.ds(0, rem)],
                        acc.at[pl.ds(base + nfull * 2 * CH, rem)])
        if with_deg:
            for j in range(rows_per_tile // D):
                pltpu.sync_copy(hbig0.at[0],
                                degsh.at[pl.ds(base + j * D, D)])
            drem = rows_per_tile - (rows_per_tile // D) * D
            pltpu.sync_copy(hbig0.at[0, pl.ds(0, drem)],
                            degsh.at[pl.ds(base + (rows_per_tile // D) * D,
                                           drem)])
        plsc.subcore_barrier()

        def issue_idx(sd, j):
            sidi, gsi, gdi, hbig, isem, lsem, ssem = sd
            o = (start + j) * CH
            pltpu.async_copy(dst.at[pl.ds(o, CH)], sidi.at[0], isem)
            pltpu.async_copy(src.at[pl.ds(o, CH)], sidi.at[1], isem)

        def issue_data(sd, j):
            sidi, gsi, gdi, hbig, isem, lsem, ssem = sd
            o = (start + j) * CH
            pltpu.make_async_copy(dst.at[pl.ds(0, CH)], sidi.at[0],
                                  isem).wait()
            pltpu.make_async_copy(src.at[pl.ds(0, CH)], sidi.at[1],
                                  isem).wait()
            pltpu.async_copy(e_hbm.at[pl.ds(o, CH)], hbig.at[pl.ds(0, CH)],
                             lsem)
            pltpu.async_copy(g_hbm.at[sidi.at[1]], gsi, lsem)
            pltpu.async_copy(g_hbm.at[sidi.at[0]], gdi, lsem)

        def wait_set(sd):
            sidi, gsi, gdi, hbig, isem, lsem, ssem = sd
            pltpu.make_async_copy(e_hbm.at[pl.ds(0, CH)],
                                  hbig.at[pl.ds(0, CH)], lsem).wait()
            pltpu.make_async_copy(g_hbm.at[sidi.at[1]], gsi, lsem).wait()
            pltpu.make_async_copy(g_hbm.at[sidi.at[0]], gdi, lsem).wait()

        def scatter(sd):
            sidi, gsi, gdi, hbig, isem, lsem, ssem = sd
            pltpu.async_copy(hbig.at[pl.ds(0, CH)], acc.at[sidi.at[0]],
                             ssem, add=True)
            pltpu.async_copy(hbig.at[pl.ds(CH, CH)], acc.at[sidi.at[1]],
                             ssem, add=True)
            if with_deg:
                pltpu.async_copy(ones, degsh.at[sidi.at[0]], ssem, add=True)
                pltpu.async_copy(ones, degsh.at[sidi.at[1]], ssem, add=True)

        def drain_scatter(sd):
            sidi, gsi, gdi, hbig, isem, lsem, ssem = sd
            pltpu.make_async_copy(hbig.at[pl.ds(0, CH)], acc.at[sidi.at[0]],
                                  ssem).wait()
            pltpu.make_async_copy(hbig.at[pl.ds(CH, CH)], acc.at[sidi.at[1]],
                                  ssem).wait()
            if with_deg:
                pltpu.make_async_copy(ones, degsh.at[sidi.at[0]],
                                      ssem).wait()
                pltpu.make_async_copy(ones, degsh.at[sidi.at[1]],
                                      ssem).wait()

        def ln_rows(sd):
            sidi, gsi, gdi, hbig, isem, lsem, ssem = sd
            lgv = [lg[pl.ds(k * L, L)] for k in range(D // L)]
            lbv = [lb[pl.ds(k * L, L)] for k in range(D // L)]

            def one_row(r):
                xf, xb = [], []
                sf = jnp.zeros((L,), jnp.float32)
                sb = jnp.zeros((L,), jnp.float32)
                qf = jnp.zeros((L,), jnp.float32)
                qb = jnp.zeros((L,), jnp.float32)
                for k in range(D // L):
                    sl = pl.ds(k * L, L)
                    sl2 = pl.ds(D + k * L, L)
                    ev = hbig[r, sl]
                    vf = gsi[r, sl] + gdi[r, sl2] + ev
                    vb = gdi[r, sl] + gsi[r, sl2] + ev
                    xf.append(vf)
                    xb.append(vb)
                    sf = sf + vf
                    sb = sb + vb
                    qf = qf + vf * vf
                    qb = qb + vb * vb
                muf = _hsum16(sf) * (1.0 / D)
                mub = _hsum16(sb) * (1.0 / D)
                vf_ = _hsum16(qf) * (1.0 / D) - muf * muf
                vb_ = _hsum16(qb) * (1.0 / D) - mub * mub
                invf = _rsqrt_vec(vf_ + EPS)
                invb = _rsqrt_vec(vb_ + EPS)
                for k in range(D // L):
                    sl = pl.ds(k * L, L)
                    yf = (xf[k] - muf) * invf * lgv[k] + lbv[k]
                    yb = (xb[k] - mub) * invb * lgv[k] + lbv[k]
                    hbig[r, sl] = jnp.maximum(yf, 0.1 * yf)
                    hbig[CH + r, sl] = jnp.maximum(yb, 0.1 * yb)

            def row_body(r, carry):
                one_row(2 * r)
                one_row(2 * r + 1)
                return carry
            lax.fori_loop(0, CH // 2, row_body, 0)

        issue_idx(sets[0], 0)
        issue_idx(sets[1], 1)
        issue_data(sets[0], 0)

        def tri_body(t, carry):
            for b in range(3):
                j = 3 * t + b
                sd = sets[b]
                wait_set(sd)

                @pl.when(j + 1 < NT)
                def _():
                    issue_data(sets[(b + 1) % 3], j + 1)
                ln_rows(sd)

                @pl.when(j > 0)
                def _():
                    drain_scatter(sets[(b + 2) % 3])

                @pl.when(j + 2 < NT)
                def _():
                    issue_idx(sets[(b + 2) % 3], j + 2)
                scatter(sd)
            return carry
        lax.fori_loop(0, NT // 3, tri_body, 0)
        drain_scatter(sets[(NT - 1) % 3])
        plsc.subcore_barrier()

        for j in range(nfull):
            pltpu.sync_copy(acc.at[pl.ds(base + j * 2 * CH, 2 * CH)],
                            s_out.at[c, pl.ds(base + j * 2 * CH, 2 * CH)])
        pltpu.sync_copy(acc.at[pl.ds(base + nfull * 2 * CH, rem)],
                        s_out.at[c, pl.ds(base + nfull * 2 * CH, rem)])
        if with_deg:
            @pl.when(s == 0)
            def _():
                pltpu.sync_copy(degsh, deg_out.at[c])

    return pl.kernel(body, out_type=out_type, mesh=mesh, scratch_types=scratch)


_msg_kernel_deg = _make_msg_kernel(True)
_msg_kernel = _make_msg_kernel(False)


def kernel(node_features, edge_list, edge_features, max_nodes, params):
    nf = node_features.astype(jnp.float32)
    n = nf.shape[0]
    nf = jnp.where(jnp.arange(n)[:, None] < max_nodes, nf, 0.0)
    npad_e = NEPAD - N_EDGES
    pad_idx = (N_NODES + jnp.arange(npad_e, dtype=jnp.int32)
               % (NPAD - N_NODES))
    src = jnp.concatenate([edge_list[:, 0].astype(jnp.int32), pad_idx])
    dst = jnp.concatenate([edge_list[:, 1].astype(jnp.int32), pad_idx])
    efp = jnp.concatenate(
        [edge_features,
         jnp.zeros((npad_e, edge_features.shape[1]), edge_features.dtype)])
    p = params
    m1, m2 = p["msg"]
    ag = p["agg"]
    w1s1, w1e1, w1d1 = m1["W1"][:D], m1["W1"][D:2 * D], m1["W1"][2 * D:]
    w1s2, w1e2, w1d2 = m2["W1"][:D], m2["W1"][D:2 * D], m2["W1"][2 * D:]

    emb0, ab1 = _node_enc_call(nf, p["node_enc"], w1s1, w1d1)
    e1, e2 = _edge_enc_call(efp, p["edge_enc"],
                            w1e1, m1["b1"], w1e2, m2["b1"])

    s1, degp = _msg_kernel_deg(src, dst, ab1, e1, m1["lng"], m1["lnb"])
    s1 = s1[:, :N_NODES, :]
    deg3 = degp[:, :N_NODES, None]
    emb1, ab2 = _combine_call(emb0, s1, deg3, m1, ag, (w1s2, w1d2))

    (s2,) = _msg_kernel(src, dst, ab2, e2, m2["lng"], m2["lnb"])
    s2 = s2[:, :N_NODES, :]
    return _combine_call(emb1, s2, deg3, m2, ag, None)


# speedup vs baseline: 1.0042x; 1.0042x over previous
import math

import jax
import jax.numpy as jnp
from jax import lax
from jax.experimental import pallas as pl
from jax.experimental.pallas import tpu as pltpu
from jax.experimental.pallas import tpu_sc as plsc

N_NODES = 10000
N_EDGES = 160000
D = 128
NPAD = 10112
CH = 16
NC, NS, L = 2, 16, 16
NW = NC * NS
NEPAD = 161280
NCHUNK = NEPAD // CH
NT = NCHUNK // NW
BN_SCALE = float(1.0 / math.sqrt(1.0 + 1e-5))
EPS = 1e-5

NB = 1000
EB = 4032


def _leaky(x):
    return jnp.maximum(x, 0.1 * x)


def _dot(a, b):
    return lax.dot_general(a, b, (((1,), (0,)), ((), ())),
                           preferred_element_type=jnp.float32)


def _full(shape):
    return pl.BlockSpec(shape, lambda i: (0,) * len(shape))


def _enc2(x, w1, b1, g1, be1, w2, b2, g2, be2):
    h = _dot(x, w1) + b1
    h = _leaky(h * (g1 * BN_SCALE) + be1)
    h = _dot(h, w2) + b2
    return _leaky(h * (g2 * BN_SCALE) + be2)


def _node_enc_body(x_ref, w1, b1, g1, be1, w2, b2, g2, be2, w1s, w1d,
                   emb_ref, ab_ref):
    h = _enc2(x_ref[...], w1[...], b1[...], g1[...], be1[...],
              w2[...], b2[...], g2[...], be2[...])
    emb_ref[...] = h
    ab_ref[:, :D] = _dot(h, w1s[...])
    ab_ref[:, D:] = _dot(h, w1d[...])


def _node_enc_call(x, ne, w1s, w1d):
    grid = (N_NODES // NB,)
    return pl.pallas_call(
        _node_enc_body,
        grid=grid,
        in_specs=[pl.BlockSpec((NB, D), lambda i: (i, 0)),
                  _full((D, D)), _full((1, D)), _full((1, D)), _full((1, D)),
                  _full((D, D)), _full((1, D)), _full((1, D)), _full((1, D)),
                  _full((D, D)), _full((D, D))],
        out_specs=[pl.BlockSpec((NB, D), lambda i: (i, 0)),
                   pl.BlockSpec((NB, 2 * D), lambda i: (i, 0))],
        out_shape=[jax.ShapeDtypeStruct((N_NODES, D), jnp.float32),
                   jax.ShapeDtypeStruct((N_NODES, 2 * D), jnp.float32)],
    )(x, ne["W1"], ne["b1"].reshape(1, D), ne["g1"].reshape(1, D),
      ne["be1"].reshape(1, D), ne["W2"], ne["b2"].reshape(1, D),
      ne["g2"].reshape(1, D), ne["be2"].reshape(1, D), w1s, w1d)


def _edge_enc_body(x_ref, w1, b1, g1, be1, w2, b2, g2, be2,
                   w1e1, c1, w1e2, c2, e1_ref, e2_ref):
    h = _enc2(x_ref[...], w1[...], b1[...], g1[...], be1[...],
              w2[...], b2[...], g2[...], be2[...])
    e1_ref[...] = _dot(h, w1e1[...]) + c1[...]
    e2_ref[...] = _dot(h, w1e2[...]) + c2[...]


def _edge_enc_call(ef, ee, w1e1, c1, w1e2, c2):
    ed = ef.shape[1]
    ne = ef.shape[0]
    grid = (ne // EB,)
    return pl.pallas_call(
        _edge_enc_body,
        grid=grid,
        in_specs=[pl.BlockSpec((EB, ed), lambda i: (i, 0)),
                  _full((ed, D)), _full((1, D)), _full((1, D)), _full((1, D)),
                  _full((D, D)), _full((1, D)), _full((1, D)), _full((1, D)),
                  _full((D, D)), _full((1, D)), _full((D, D)), _full((1, D))],
        out_specs=[pl.BlockSpec((EB, D), lambda i: (i, 0))] * 2,
        out_shape=[jax.ShapeDtypeStruct((ne, D), jnp.float32)] * 2,
    )(ef, ee["W1"], ee["b1"].reshape(1, D), ee["g1"].reshape(1, D),
      ee["be1"].reshape(1, D), ee["W2"], ee["b2"].reshape(1, D),
      ee["g2"].reshape(1, D), ee["be2"].reshape(1, D),
      w1e1, c1.reshape(1, D), w1e2, c2.reshape(1, D))


def _combine_body_proj(emb_ref, s_ref, deg_ref, w2m, b2m, wa1, ba1, lg, lb,
                       wa2, ba2, w1s, w1d, out_ref, ab_ref):
    e = emb_ref[...]
    s = s_ref[0, :, :] + s_ref[1, :, :]
    deg = deg_ref[0, :, :] + deg_ref[1, :, :]
    new = e + _dot(s, w2m[...]) + deg * b2m[...]
    g = _dot(new, wa1[...]) + ba1[...]
    mu = jnp.mean(g, axis=-1, keepdims=True)
    var = jnp.mean((g - mu) ** 2, axis=-1, keepdims=True)
    h = _leaky((g - mu) * lax.rsqrt(var + EPS) * lg[...] + lb[...])
    out = e + _dot(h, wa2[...]) + ba2[...]
    out_ref[...] = out
    ab_ref[:, :D] = _dot(out, w1s[...])
    ab_ref[:, D:] = _dot(out, w1d[...])


def _combine_body_final(emb_ref, s_ref, deg_ref, w2m, b2m, wa1, ba1, lg, lb,
                        wa2, ba2, out_ref):
    e = emb_ref[...]
    s = s_ref[0, :, :] + s_ref[1, :, :]
    deg = deg_ref[0, :, :] + deg_ref[1, :, :]
    new = e + _dot(s, w2m[...]) + deg * b2m[...]
    g = _dot(new, wa1[...]) + ba1[...]
    mu = jnp.mean(g, axis=-1, keepdims=True)
    var = jnp.mean((g - mu) ** 2, axis=-1, keepdims=True)
    h = _leaky((g - mu) * lax.rsqrt(var + EPS) * lg[...] + lb[...])
    out_ref[...] = e + _dot(h, wa2[...]) + ba2[...]


def _combine_call(emb, S, degp, mp, ag, proj):
    grid = (N_NODES // NB,)
    in_specs = [pl.BlockSpec((NB, D), lambda i: (i, 0)),
                pl.BlockSpec((2, NB, D), lambda i: (0, i, 0)),
                pl.BlockSpec((2, NB, 1), lambda i: (0, i, 0)),
                _full((D, D)), _full((1, D)),
                _full((D, D)), _full((1, D)), _full((1, D)), _full((1, D)),
                _full((D, D)), _full((1, D))]
    args = [emb, S, degp, mp["W2"], mp["b2"].reshape(1, D),
            ag["W1"], ag["b1"].reshape(1, D), ag["lng"].reshape(1, D),
            ag["lnb"].reshape(1, D), ag["W2"], ag["b2"].reshape(1, D)]
    if proj is not None:
        w1s, w1d = proj
        return pl.pallas_call(
            _combine_body_proj,
            grid=grid,
            in_specs=in_specs + [_full((D, D)), _full((D, D))],
            out_specs=[pl.BlockSpec((NB, D), lambda i: (i, 0)),
                       pl.BlockSpec((NB, 2 * D), lambda i: (i, 0))],
            out_shape=[jax.ShapeDtypeStruct((N_NODES, D), jnp.float32),
                       jax.ShapeDtypeStruct((N_NODES, 2 * D), jnp.float32)],
        )(*args, w1s, w1d)
    return pl.pallas_call(
        _combine_body_final,
        grid=grid,
        in_specs=in_specs,
        out_specs=pl.BlockSpec((NB, D), lambda i: (i, 0)),
        out_shape=jax.ShapeDtypeStruct((N_NODES, D), jnp.float32),
    )(*args)


_GDN = lax.GatherDimensionNumbers(offset_dims=(), collapsed_slice_dims=(0,),
                                  start_index_map=(0,))


def _perm16(v, idx):
    return lax.gather(v, idx[:, None], _GDN, slice_sizes=(1,),
                      mode=lax.GatherScatterMode.PROMISE_IN_BOUNDS)


def _hsum16(v):
    lanes = lax.iota(jnp.int32, 16)
    for sh in (8, 4, 2, 1):
        v = v + _perm16(v, lanes ^ sh)
    return v


def _rsqrt_vec(x):
    i = lax.bitcast_convert_type(x, jnp.int32)
    y = lax.bitcast_convert_type(jnp.int32(0x5F3759DF) - (i >> 1), jnp.float32)
    for _ in range(2):
        y = y * (1.5 - 0.5 * x * y * y)
    return y


def _make_msg_kernel(with_deg):
    mesh = plsc.VectorSubcoreMesh(core_axis_name="c", subcore_axis_name="s")
    out_type = [jax.ShapeDtypeStruct((NC, NPAD, D), jnp.float32)]
    if with_deg:
        out_type.append(jax.ShapeDtypeStruct((NC, NPAD), jnp.float32))
    scratch = [
        pltpu.VMEM((2, CH), jnp.int32),
        pltpu.VMEM((2, CH), jnp.int32),
        pltpu.VMEM((2, CH), jnp.int32),
        pltpu.VMEM((CH, 2 * D), jnp.float32),
        pltpu.VMEM((CH, 2 * D), jnp.float32),
        pltpu.VMEM((CH, 2 * D), jnp.float32),
        pltpu.VMEM((CH, 2 * D), jnp.float32),
        pltpu.VMEM((CH, 2 * D), jnp.float32),
        pltpu.VMEM((CH, 2 * D), jnp.float32),
        pltpu.VMEM((2 * CH, D), jnp.float32),
        pltpu.VMEM((2 * CH, D), jnp.float32),
        pltpu.VMEM((2 * CH, D), jnp.float32),
        pltpu.VMEM((CH,), jnp.float32),
        pltpu.VMEM((D,), jnp.float32),
        pltpu.VMEM((D,), jnp.float32),
        pltpu.VMEM_SHARED((NPAD, D), jnp.float32),
        pltpu.VMEM_SHARED((NPAD,), jnp.float32),
        pltpu.SemaphoreType.DMA,
        pltpu.SemaphoreType.DMA,
        pltpu.SemaphoreType.DMA,
        pltpu.SemaphoreType.DMA,
        pltpu.SemaphoreType.DMA,
        pltpu.SemaphoreType.DMA,
        pltpu.SemaphoreType.DMA,
        pltpu.SemaphoreType.DMA,
        pltpu.SemaphoreType.DMA,
    ]

    def body(src, dst, g_hbm, e_hbm, lg_hbm, lb_hbm, s_out, *rest):
        if with_deg:
            deg_out = rest[0]
            rest = rest[1:]
        else:
            deg_out = None
        (sidi0, sidi1, sidi2, gsi0, gsi1, gsi2, gdi0, gdi1, gdi2,
         hbig0, hbig1, hbig2, ones, lg, lb, acc, degsh,
         im0, im1, im2, lm0, lm1, lm2, sm0, sm1, sm2) = rest
        sets = [(sidi0, gsi0, gdi0, hbig0, im0, lm0, sm0),
                (sidi1, gsi1, gdi1, hbig1, im1, lm1, sm1),
                (sidi2, gsi2, gdi2, hbig2, im2, lm2, sm2)]
        c = lax.axis_index("c")
        s = lax.axis_index("s")
        w = s * NC + c
        start = w * NT

        pltpu.sync_copy(lg_hbm, lg)
        pltpu.sync_copy(lb_hbm, lb)
        lgv = [lg[pl.ds(k * L, L)] for k in range(D // L)]
        lbv = [lb[pl.ds(k * L, L)] for k in range(D // L)]

        def zrow(r, carry):
            for k in range(D // L):
                hbig0[r, pl.ds(k * L, L)] = jnp.zeros((L,), jnp.float32)
            return carry
        lax.fori_loop(0, 2 * CH, zrow, 0)
        for k in range(CH // L):
            ones[pl.ds(k * L, L)] = jnp.ones((L,), jnp.float32)

        rows_per_tile = NPAD // NS
        base = s * rows_per_tile
        nfull = rows_per_tile // (2 * CH)
        rem = rows_per_tile - nfull * 2 * CH
        for j in range(nfull):
            pltpu.sync_copy(hbig0, acc.at[pl.ds(base + j * 2 * CH, 2 * CH)])
        pltpu.sync_copy(hbig0.at[pl.ds(0, rem)],
                        acc.at[pl.ds(base + nfull * 2 * CH, rem)])
        if with_deg:
            for j in range(rows_per_tile // D):
                pltpu.sync_copy(hbig0.at[0],
                                degsh.at[pl.ds(base + j * D, D)])
            drem = rows_per_tile - (rows_per_tile // D) * D
            pltpu.sync_copy(hbig0.at[0, pl.ds(0, drem)],
                            degsh.at[pl.ds(base + (rows_per_tile // D) * D,
                                           drem)])
        plsc.subcore_barrier()

        def issue_idx(sd, j):
            sidi, gsi, gdi, hbig, isem, lsem, ssem = sd
            o = (start + j) * CH
            pltpu.async_copy(dst.at[pl.ds(o, CH)], sidi.at[0], isem)
            pltpu.async_copy(src.at[pl.ds(o, CH)], sidi.at[1], isem)

        def issue_data(sd, j):
            sidi, gsi, gdi, hbig, isem, lsem, ssem = sd
            o = (start + j) * CH
            pltpu.make_async_copy(dst.at[pl.ds(0, CH)], sidi.at[0],
                                  isem).wait()
            pltpu.make_async_copy(src.at[pl.ds(0, CH)], sidi.at[1],
                                  isem).wait()
            pltpu.async_copy(e_hbm.at[pl.ds(o, CH)], hbig.at[pl.ds(0, CH)],
                             lsem)
            pltpu.async_copy(g_hbm.at[sidi.at[1]], gsi, lsem)
            pltpu.async_copy(g_hbm.at[sidi.at[0]], gdi, lsem)

        def wait_set(sd):
            sidi, gsi, gdi, hbig, isem, lsem, ssem = sd
            pltpu.make_async_copy(e_hbm.at[pl.ds(0, CH)],
                                  hbig.at[pl.ds(0, CH)], lsem).wait()
            pltpu.make_async_copy(g_hbm.at[sidi.at[1]], gsi, lsem).wait()
            pltpu.make_async_copy(g_hbm.at[sidi.at[0]], gdi, lsem).wait()

        def scatter(sd):
            sidi, gsi, gdi, hbig, isem, lsem, ssem = sd
            pltpu.async_copy(hbig.at[pl.ds(0, CH)], acc.at[sidi.at[0]],
                             ssem, add=True)
            pltpu.async_copy(hbig.at[pl.ds(CH, CH)], acc.at[sidi.at[1]],
                             ssem, add=True)
            if with_deg:
                pltpu.async_copy(ones, degsh.at[sidi.at[0]], ssem, add=True)
                pltpu.async_copy(ones, degsh.at[sidi.at[1]], ssem, add=True)

        def drain_scatter(sd):
            sidi, gsi, gdi, hbig, isem, lsem, ssem = sd
            pltpu.make_async_copy(hbig.at[pl.ds(0, CH)], acc.at[sidi.at[0]],
                                  ssem).wait()
            pltpu.make_async_copy(hbig.at[pl.ds(CH, CH)], acc.at[sidi.at[1]],
                                  ssem).wait()
            if with_deg:
                pltpu.make_async_copy(ones, degsh.at[sidi.at[0]],
                                      ssem).wait()
                pltpu.make_async_copy(ones, degsh.at[sidi.at[1]],
                                      ssem).wait()

        def ln_rows(sd):
            sidi, gsi, gdi, hbig, isem, lsem, ssem = sd

            def one_row(r):
                xf, xb = [], []
                sf = jnp.zeros((L,), jnp.float32)
                sb = jnp.zeros((L,), jnp.float32)
                qf = jnp.zeros((L,), jnp.float32)
                qb = jnp.zeros((L,), jnp.float32)
                for k in range(D // L):
                    sl = pl.ds(k * L, L)
                    sl2 = pl.ds(D + k * L, L)
                    ev = hbig[r, sl]
                    vf = gsi[r, sl] + gdi[r, sl2] + ev
                    vb = gdi[r, sl] + gsi[r, sl2] + ev
                    xf.append(vf)
                    xb.append(vb)
                    sf = sf + vf
                    sb = sb + vb
                    qf = qf + vf * vf
                    qb = qb + vb * vb
                muf = _hsum16(sf) * (1.0 / D)
                mub = _hsum16(sb) * (1.0 / D)
                vf_ = _hsum16(qf) * (1.0 / D) - muf * muf
                vb_ = _hsum16(qb) * (1.0 / D) - mub * mub
                invf = _rsqrt_vec(vf_ + EPS)
                invb = _rsqrt_vec(vb_ + EPS)
                for k in range(D // L):
                    sl = pl.ds(k * L, L)
                    yf = (xf[k] - muf) * invf * lgv[k] + lbv[k]
                    yb = (xb[k] - mub) * invb * lgv[k] + lbv[k]
                    hbig[r, sl] = jnp.maximum(yf, 0.1 * yf)
                    hbig[CH + r, sl] = jnp.maximum(yb, 0.1 * yb)

            def row_body(r, carry):
                one_row(2 * r)
                one_row(2 * r + 1)
                return carry
            lax.fori_loop(0, CH // 2, row_body, 0)

        issue_idx(sets[0], 0)
        issue_idx(sets[1], 1)
        issue_data(sets[0], 0)

        def tri_body(t, carry):
            for b in range(3):
                j = 3 * t + b
                sd = sets[b]
                wait_set(sd)

                @pl.when(j + 1 < NT)
                def _():
                    issue_data(sets[(b + 1) % 3], j + 1)
                ln_rows(sd)

                @pl.when(j > 0)
                def _():
                    drain_scatter(sets[(b + 2) % 3])

                @pl.when(j + 2 < NT)
                def _():
                    issue_idx(sets[(b + 2) % 3], j + 2)
                scatter(sd)
            return carry
        lax.fori_loop(0, NT // 3, tri_body, 0)
        drain_scatter(sets[(NT - 1) % 3])
        plsc.subcore_barrier()

        for j in range(nfull):
            pltpu.sync_copy(acc.at[pl.ds(base + j * 2 * CH, 2 * CH)],
                            s_out.at[c, pl.ds(base + j * 2 * CH, 2 * CH)])
        pltpu.sync_copy(acc.at[pl.ds(base + nfull * 2 * CH, rem)],
                        s_out.at[c, pl.ds(base + nfull * 2 * CH, rem)])
        if with_deg:
            @pl.when(s == 0)
            def _():
                pltpu.sync_copy(degsh, deg_out.at[c])

    return pl.kernel(body, out_type=out_type, mesh=mesh, scratch_types=scratch)


_msg_kernel_deg = _make_msg_kernel(True)
_msg_kernel = _make_msg_kernel(False)


def kernel(node_features, edge_list, edge_features, max_nodes, params):
    nf = node_features.astype(jnp.float32)
    n = nf.shape[0]
    nf = jnp.where(jnp.arange(n)[:, None] < max_nodes, nf, 0.0)
    npad_e = NEPAD - N_EDGES
    pad_idx = (N_NODES + jnp.arange(npad_e, dtype=jnp.int32)
               % (NPAD - N_NODES))
    src = jnp.concatenate([edge_list[:, 0].astype(jnp.int32), pad_idx])
    dst = jnp.concatenate([edge_list[:, 1].astype(jnp.int32), pad_idx])
    efp = jnp.concatenate(
        [edge_features,
         jnp.zeros((npad_e, edge_features.shape[1]), edge_features.dtype)])
    p = params
    m1, m2 = p["msg"]
    ag = p["agg"]
    w1s1, w1e1, w1d1 = m1["W1"][:D], m1["W1"][D:2 * D], m1["W1"][2 * D:]
    w1s2, w1e2, w1d2 = m2["W1"][:D], m2["W1"][D:2 * D], m2["W1"][2 * D:]

    emb0, ab1 = _node_enc_call(nf, p["node_enc"], w1s1, w1d1)
    e1, e2 = _edge_enc_call(efp, p["edge_enc"],
                            w1e1, m1["b1"], w1e2, m2["b1"])

    s1, degp = _msg_kernel_deg(src, dst, ab1, e1, m1["lng"], m1["lnb"])
    s1 = s1[:, :N_NODES, :]
    deg3 = degp[:, :N_NODES, None]
    emb1, ab2 = _combine_call(emb0, s1, deg3, m1, ag, (w1s2, w1d2))

    (s2,) = _msg_kernel(src, dst, ab2, e2, m2["lng"], m2["lnb"])
    s2 = s2[:, :N_NODES, :]
    return _combine_call(emb1, s2, deg3, m2, ag, None)
